# baseline (speedup 1.0000x reference)
import functools

import jax
import jax.numpy as jnp
from jax import lax
from jax.experimental import pallas as pl
from jax.experimental.pallas import tpu as pltpu
from jax.experimental.pallas import tpu_sc as plsc

N_EMBED = 8192
EMBED_DIM = 64
M_TOTAL = 8192
BLOCK_M = 1024
GRID_M = M_TOTAL // BLOCK_M


def _normalize(x):
    n = jnp.sqrt(jnp.sum(x * x, axis=-1, keepdims=True))
    return x / jnp.maximum(n, 1e-12)


def _tc_body(z_ref, e_ref, idx_ref, en_ref, loss_ref, en_vmem, acc_ref):
    i = pl.program_id(0)

    @pl.when(i == 0)
    def _():
        en0 = _normalize(e_ref[...])
        en_vmem[...] = en0
        en_ref[...] = jnp.concatenate(
            [en0, jnp.zeros((N_EMBED, 128 - EMBED_DIM), jnp.float32)], axis=1)
        acc_ref[...] = jnp.zeros_like(acc_ref)

    zn = _normalize(z_ref[...].reshape(BLOCK_M, EMBED_DIM))
    CHUNK = 512
    n_grp = N_EMBED // 128
    runmax = jnp.full((BLOCK_M, 128), -jnp.inf, jnp.float32)
    runidx = jnp.zeros((BLOCK_M, 128), jnp.int32)
    for kc in range(N_EMBED // CHUNK):
        sim = lax.dot_general(
            zn, en_vmem[kc * CHUNK:(kc + 1) * CHUNK, :],
            dimension_numbers=(((1,), (1,)), ((), ())),
            preferred_element_type=jnp.float32,
        )
        for cc in range(CHUNK // 128):
            c = kc * (CHUNK // 128) + cc
            v = sim[:, cc * 128:(cc + 1) * 128]
            m = v > runmax
            runidx = jnp.where(m, c, runidx)
            runmax = jnp.where(m, v, runmax)
    maxv = jnp.max(runmax, axis=1)
    lane = lax.broadcasted_iota(jnp.int32, (BLOCK_M, 128), 1)
    cand = jnp.where(runmax == maxv[:, None], runidx * 128 + lane, N_EMBED)
    idx = jnp.min(cand, axis=1)
    idx_ref[0, 0, :] = idx

    acc_ref[...] += maxv.reshape(acc_ref.shape)

    @pl.when(i == GRID_M - 1)
    def _():
        loss_ref[0] = 2.5 * (1.0 - jnp.sum(acc_ref[...]) / M_TOTAL)


def _tc_search(z_flat, e):
    return pl.pallas_call(
        _tc_body,
        grid=(GRID_M,),
        in_specs=[
            pl.BlockSpec((1, BLOCK_M, EMBED_DIM), lambda i: (i, 0, 0)),
            pl.BlockSpec((N_EMBED, EMBED_DIM), lambda i: (0, 0)),
        ],
        out_specs=[
            pl.BlockSpec((1, 1, BLOCK_M), lambda i: (i, 0, 0)),
            pl.BlockSpec((N_EMBED, 128), lambda i: (0, 0)),
            pl.BlockSpec(memory_space=pltpu.SMEM),
        ],
        out_shape=[
            jax.ShapeDtypeStruct((GRID_M, 1, BLOCK_M), jnp.int32),
            jax.ShapeDtypeStruct((N_EMBED, 128), jnp.float32),
            jax.ShapeDtypeStruct((1,), jnp.float32),
        ],
        scratch_shapes=[
            pltpu.VMEM((N_EMBED, EMBED_DIM), jnp.float32),
            pltpu.VMEM((BLOCK_M // 128, 128), jnp.float32),
        ],
    )(z_flat, e)


_SC_INFO = plsc.get_sparse_core_info()
_NC, _NS = _SC_INFO.num_cores, _SC_INFO.num_subcores
_NW = _NC * _NS
_B_PER_W = M_TOTAL // _NW


_W_PER_G = _NW // GRID_M


@functools.partial(
    pl.kernel,
    mesh=plsc.VectorSubcoreMesh(core_axis_name="c", subcore_axis_name="s"),
    out_type=jax.ShapeDtypeStruct((M_TOTAL, 128), jnp.float32),
    scratch_types=[
        pltpu.VMEM((_B_PER_W,), jnp.int32),
        pltpu.VMEM((_B_PER_W, 128), jnp.float32),
        pltpu.SemaphoreType.DMA,
    ],
)
def _sc_gather(table_hbm, idx_hbm, out_hbm, idx_v, rows_v, sem):
    wid = lax.axis_index("s") * _NC + lax.axis_index("c")
    g = wid // _W_PER_G
    off = (wid % _W_PER_G) * _B_PER_W
    pltpu.sync_copy(idx_hbm.at[g, 0, pl.ds(off, _B_PER_W)], idx_v)
    pltpu.async_copy(table_hbm.at[idx_v], rows_v, sem).wait()
    pltpu.sync_copy(rows_v, out_hbm.at[pl.ds(g * BLOCK_M + off, _B_PER_W)])


def kernel(z, embedding_weight):
    idx3, en, loss = _tc_search(z, embedding_weight)
    zq_pad = _sc_gather(en, idx3)
    return zq_pad[:, :EMBED_DIM].reshape(z.shape), loss.reshape(())


# speedup vs baseline: 1.0345x; 1.0345x over previous
import functools

import jax
import jax.numpy as jnp
from jax import lax
from jax.experimental import pallas as pl
from jax.experimental.pallas import tpu as pltpu
from jax.experimental.pallas import tpu_sc as plsc

N_EMBED = 8192
EMBED_DIM = 64
M_TOTAL = 8192
BLOCK_M = 1024
GRID_M = M_TOTAL // BLOCK_M


def _normalize(x):
    n = jnp.sqrt(jnp.sum(x * x, axis=-1, keepdims=True))
    return x / jnp.maximum(n, 1e-12)


def _tc_body(z_ref, e_ref, idx_ref, en_ref, loss_ref, en_vmem, acc_ref):
    i = pl.program_id(0)

    @pl.when(i == 0)
    def _():
        en0 = _normalize(e_ref[...].T)
        en_vmem[...] = en0
        en_ref[...] = jnp.concatenate(
            [en0, jnp.zeros((N_EMBED, 128 - EMBED_DIM), jnp.float32)], axis=1)
        acc_ref[...] = jnp.zeros_like(acc_ref)

    zn = _normalize(z_ref[...].reshape(EMBED_DIM, BLOCK_M).T)
    sim = lax.dot_general(
        zn, en_vmem[...],
        dimension_numbers=(((1,), (1,)), ((), ())),
        preferred_element_type=jnp.float32,
    )
    n_grp = N_EMBED // 128
    runmax = sim[:, 0:128]
    runidx = jnp.zeros((BLOCK_M, 128), jnp.int32)
    for c in range(1, n_grp):
        v = sim[:, c * 128:(c + 1) * 128]
        m = v > runmax
        runidx = jnp.where(m, c, runidx)
        runmax = jnp.where(m, v, runmax)
    maxv = jnp.max(runmax, axis=1)
    lane = lax.broadcasted_iota(jnp.int32, (BLOCK_M, 128), 1)
    cand = jnp.where(runmax == maxv[:, None], runidx * 128 + lane, N_EMBED)
    idx = jnp.min(cand, axis=1)
    idx_ref[0, 0, :] = idx

    acc_ref[...] += maxv.reshape(acc_ref.shape)

    @pl.when(i == GRID_M - 1)
    def _():
        loss_ref[0] = 2.5 * (1.0 - jnp.sum(acc_ref[...]) / M_TOTAL)


def _tc_search(z_flat, e):
    return pl.pallas_call(
        _tc_body,
        grid=(GRID_M,),
        in_specs=[
            pl.BlockSpec((1, EMBED_DIM, BLOCK_M), lambda i: (i, 0, 0)),
            pl.BlockSpec((EMBED_DIM, N_EMBED), lambda i: (0, 0)),
        ],
        out_specs=[
            pl.BlockSpec((1, 1, BLOCK_M), lambda i: (i, 0, 0)),
            pl.BlockSpec((N_EMBED, 128), lambda i: (0, 0)),
            pl.BlockSpec(memory_space=pltpu.SMEM),
        ],
        out_shape=[
            jax.ShapeDtypeStruct((GRID_M, 1, BLOCK_M), jnp.int32),
            jax.ShapeDtypeStruct((N_EMBED, 128), jnp.float32),
            jax.ShapeDtypeStruct((1,), jnp.float32),
        ],
        scratch_shapes=[
            pltpu.VMEM((N_EMBED, EMBED_DIM), jnp.float32),
            pltpu.VMEM((BLOCK_M // 128, 128), jnp.float32),
        ],
    )(z_flat, e)


_SC_INFO = plsc.get_sparse_core_info()
_NC, _NS = _SC_INFO.num_cores, _SC_INFO.num_subcores
_NW = _NC * _NS
_B_PER_W = M_TOTAL // _NW


_W_PER_G = _NW // GRID_M


@functools.partial(
    pl.kernel,
    mesh=plsc.VectorSubcoreMesh(core_axis_name="c", subcore_axis_name="s"),
    out_type=jax.ShapeDtypeStruct((M_TOTAL, 128), jnp.float32),
    scratch_types=[
        pltpu.VMEM((_B_PER_W,), jnp.int32),
        pltpu.VMEM((_B_PER_W, 128), jnp.float32),
        pltpu.SemaphoreType.DMA,
    ],
)
def _sc_gather(table_hbm, idx_hbm, out_hbm, idx_v, rows_v, sem):
    wid = lax.axis_index("s") * _NC + lax.axis_index("c")
    g = wid // _W_PER_G
    off = (wid % _W_PER_G) * _B_PER_W
    pltpu.sync_copy(idx_hbm.at[g, 0, pl.ds(off, _B_PER_W)], idx_v)
    pltpu.async_copy(table_hbm.at[idx_v], rows_v, sem).wait()
    pltpu.sync_copy(rows_v, out_hbm.at[pl.ds(g * BLOCK_M + off, _B_PER_W)])


def kernel(z, embedding_weight):
    zt = jnp.transpose(z, (0, 2, 1))
    et = embedding_weight.T
    idx3, en, loss = _tc_search(zt, et)
    zq_pad = _sc_gather(en, idx3)
    return zq_pad[:, :EMBED_DIM].reshape(z.shape), loss.reshape(())


# speedup vs baseline: 1.0800x; 1.0439x over previous
import functools

import jax
import jax.numpy as jnp
from jax import lax
from jax.experimental import pallas as pl
from jax.experimental.pallas import tpu as pltpu
from jax.experimental.pallas import tpu_sc as plsc

N_EMBED = 8192
EMBED_DIM = 64
M_TOTAL = 8192
BLOCK_M = 1024
GRID_M = M_TOTAL // BLOCK_M


def _normalize(x):
    n = jnp.sqrt(jnp.sum(x * x, axis=-1, keepdims=True))
    return x / jnp.maximum(n, 1e-12)


def _tc_body(z_ref, e_ref, idx_ref, en_ref, loss_ref, en_vmem, acc_ref):
    i = pl.program_id(0)

    @pl.when(i == 0)
    def _():
        en0 = _normalize(e_ref[...].T)
        en_vmem[...] = en0
        en_ref[...] = jnp.concatenate(
            [en0, jnp.zeros((N_EMBED, 128 - EMBED_DIM), jnp.float32)], axis=1)
        acc_ref[...] = jnp.zeros_like(acc_ref)

    zt = z_ref[...].reshape(EMBED_DIM, BLOCK_M)
    n = jnp.sqrt(jnp.sum(zt * zt, axis=0, keepdims=True))
    znt = zt / jnp.maximum(n, 1e-12)
    sim = lax.dot_general(
        znt, en_vmem[...],
        dimension_numbers=(((0,), (1,)), ((), ())),
        preferred_element_type=jnp.float32,
    )
    n_grp = N_EMBED // 128
    runmax = sim[:, 0:128]
    runidx = jnp.zeros((BLOCK_M, 128), jnp.int32)
    for c in range(1, n_grp):
        v = sim[:, c * 128:(c + 1) * 128]
        m = v > runmax
        runidx = jnp.where(m, c, runidx)
        runmax = jnp.where(m, v, runmax)
    maxv = jnp.max(runmax, axis=1)
    lane = lax.broadcasted_iota(jnp.int32, (BLOCK_M, 128), 1)
    cand = jnp.where(runmax == maxv[:, None], runidx * 128 + lane, N_EMBED)
    idx = jnp.min(cand, axis=1)
    idx_ref[0, 0, :] = idx

    acc_ref[...] += maxv.reshape(acc_ref.shape)

    @pl.when(i == GRID_M - 1)
    def _():
        loss_ref[0] = 2.5 * (1.0 - jnp.sum(acc_ref[...]) / M_TOTAL)


def _tc_search(z_flat, e):
    return pl.pallas_call(
        _tc_body,
        grid=(GRID_M,),
        in_specs=[
            pl.BlockSpec((1, EMBED_DIM, BLOCK_M), lambda i: (i, 0, 0)),
            pl.BlockSpec((EMBED_DIM, N_EMBED), lambda i: (0, 0)),
        ],
        out_specs=[
            pl.BlockSpec((1, 1, BLOCK_M), lambda i: (i, 0, 0)),
            pl.BlockSpec((N_EMBED, 128), lambda i: (0, 0)),
            pl.BlockSpec(memory_space=pltpu.SMEM),
        ],
        out_shape=[
            jax.ShapeDtypeStruct((GRID_M, 1, BLOCK_M), jnp.int32),
            jax.ShapeDtypeStruct((N_EMBED, 128), jnp.float32),
            jax.ShapeDtypeStruct((1,), jnp.float32),
        ],
        scratch_shapes=[
            pltpu.VMEM((N_EMBED, EMBED_DIM), jnp.float32),
            pltpu.VMEM((BLOCK_M // 128, 128), jnp.float32),
        ],
    )(z_flat, e)


_SC_INFO = plsc.get_sparse_core_info()
_NC, _NS = _SC_INFO.num_cores, _SC_INFO.num_subcores
_NW = _NC * _NS
_B_PER_W = M_TOTAL // _NW


_W_PER_G = _NW // GRID_M


@functools.partial(
    pl.kernel,
    mesh=plsc.VectorSubcoreMesh(core_axis_name="c", subcore_axis_name="s"),
    out_type=jax.ShapeDtypeStruct((M_TOTAL, 128), jnp.float32),
    scratch_types=[
        pltpu.VMEM((_B_PER_W,), jnp.int32),
        pltpu.VMEM((_B_PER_W, 128), jnp.float32),
        pltpu.SemaphoreType.DMA,
    ],
)
def _sc_gather(table_hbm, idx_hbm, out_hbm, idx_v, rows_v, sem):
    wid = lax.axis_index("s") * _NC + lax.axis_index("c")
    g = wid // _W_PER_G
    off = (wid % _W_PER_G) * _B_PER_W
    pltpu.sync_copy(idx_hbm.at[g, 0, pl.ds(off, _B_PER_W)], idx_v)
    pltpu.async_copy(table_hbm.at[idx_v], rows_v, sem).wait()
    pltpu.sync_copy(rows_v, out_hbm.at[pl.ds(g * BLOCK_M + off, _B_PER_W)])


def kernel(z, embedding_weight):
    zt = jnp.transpose(z, (0, 2, 1))
    et = embedding_weight.T
    idx3, en, loss = _tc_search(zt, et)
    zq_pad = _sc_gather(en, idx3)
    return zq_pad[:, :EMBED_DIM].reshape(z.shape), loss.reshape(())


# speedup vs baseline: 1.1033x; 1.0217x over previous
import functools

import jax
import jax.numpy as jnp
from jax import lax
from jax.experimental import pallas as pl
from jax.experimental.pallas import tpu as pltpu
from jax.experimental.pallas import tpu_sc as plsc

N_EMBED = 8192
EMBED_DIM = 64
M_TOTAL = 8192
BLOCK_M = 1024
GRID_M = M_TOTAL // BLOCK_M


def _normalize(x):
    n = jnp.sqrt(jnp.sum(x * x, axis=-1, keepdims=True))
    return x / jnp.maximum(n, 1e-12)


def _tc_body(z_ref, e_ref, idx_ref, en_ref, loss_ref, en_vmem, acc_ref):
    i = pl.program_id(0)

    @pl.when(i == 0)
    def _():
        en0 = _normalize(e_ref[...].T)
        en_vmem[...] = en0
        en_ref[...] = jnp.concatenate(
            [en0, jnp.zeros((N_EMBED, 128 - EMBED_DIM), jnp.float32)], axis=1)
        acc_ref[...] = jnp.zeros_like(acc_ref)

    zt = z_ref[...].reshape(EMBED_DIM, BLOCK_M)
    n = jnp.sqrt(jnp.sum(zt * zt, axis=0, keepdims=True))
    znt = zt / jnp.maximum(n, 1e-12)
    sim = lax.dot_general(
        znt, en_vmem[...],
        dimension_numbers=(((0,), (1,)), ((), ())),
        preferred_element_type=jnp.float32,
    )
    n_grp = N_EMBED // 128
    STRIPE = 128
    lane = lax.broadcasted_iota(jnp.int32, (STRIPE, 128), 1)
    for r in range(BLOCK_M // STRIPE):
        rs = r * STRIPE
        runmax = sim[rs:rs + STRIPE, 0:128]
        runidx = jnp.zeros((STRIPE, 128), jnp.int32)
        for c in range(1, n_grp):
            v = sim[rs:rs + STRIPE, c * 128:(c + 1) * 128]
            m = v > runmax
            runidx = jnp.where(m, c, runidx)
            runmax = jnp.where(m, v, runmax)
        maxv = jnp.max(runmax, axis=1)
        cand = jnp.where(runmax == maxv[:, None], runidx * 128 + lane, N_EMBED)
        idx = jnp.min(cand, axis=1)
        idx_ref[0, 0, rs:rs + STRIPE] = idx
        acc_ref[r, :] += maxv

    @pl.when(i == GRID_M - 1)
    def _():
        loss_ref[0] = 2.5 * (1.0 - jnp.sum(acc_ref[...]) / M_TOTAL)


def _tc_search(z_flat, e):
    return pl.pallas_call(
        _tc_body,
        grid=(GRID_M,),
        in_specs=[
            pl.BlockSpec((1, EMBED_DIM, BLOCK_M), lambda i: (i, 0, 0)),
            pl.BlockSpec((EMBED_DIM, N_EMBED), lambda i: (0, 0)),
        ],
        out_specs=[
            pl.BlockSpec((1, 1, BLOCK_M), lambda i: (i, 0, 0)),
            pl.BlockSpec((N_EMBED, 128), lambda i: (0, 0)),
            pl.BlockSpec(memory_space=pltpu.SMEM),
        ],
        out_shape=[
            jax.ShapeDtypeStruct((GRID_M, 1, BLOCK_M), jnp.int32),
            jax.ShapeDtypeStruct((N_EMBED, 128), jnp.float32),
            jax.ShapeDtypeStruct((1,), jnp.float32),
        ],
        scratch_shapes=[
            pltpu.VMEM((N_EMBED, EMBED_DIM), jnp.float32),
            pltpu.VMEM((BLOCK_M // 128, 128), jnp.float32),
        ],
    )(z_flat, e)


_SC_INFO = plsc.get_sparse_core_info()
_NC, _NS = _SC_INFO.num_cores, _SC_INFO.num_subcores
_NW = _NC * _NS
_B_PER_W = M_TOTAL // _NW


_W_PER_G = _NW // GRID_M


@functools.partial(
    pl.kernel,
    mesh=plsc.VectorSubcoreMesh(core_axis_name="c", subcore_axis_name="s"),
    out_type=jax.ShapeDtypeStruct((M_TOTAL, 128), jnp.float32),
    scratch_types=[
        pltpu.VMEM((_B_PER_W,), jnp.int32),
        pltpu.VMEM((_B_PER_W, 128), jnp.float32),
        pltpu.SemaphoreType.DMA,
    ],
)
def _sc_gather(table_hbm, idx_hbm, out_hbm, idx_v, rows_v, sem):
    wid = lax.axis_index("s") * _NC + lax.axis_index("c")
    g = wid // _W_PER_G
    off = (wid % _W_PER_G) * _B_PER_W
    pltpu.sync_copy(idx_hbm.at[g, 0, pl.ds(off, _B_PER_W)], idx_v)
    pltpu.async_copy(table_hbm.at[idx_v], rows_v, sem).wait()
    pltpu.sync_copy(rows_v, out_hbm.at[pl.ds(g * BLOCK_M + off, _B_PER_W)])


def kernel(z, embedding_weight):
    zt = jnp.transpose(z, (0, 2, 1))
    et = embedding_weight.T
    idx3, en, loss = _tc_search(zt, et)
    zq_pad = _sc_gather(en, idx3)
    return zq_pad[:, :EMBED_DIM].reshape(z.shape), loss.reshape(())
